# baseline (speedup 1.0000x reference)
import jax
import jax.numpy as jnp
from jax.experimental import pallas as pl
import jax.experimental.pallas.tpu as pltpu


def _permute_body(x_ref, o_ref):
    lam = jax.lax.broadcasted_iota(jnp.int32, (1, 128), 1)
    m40_eq = ((lam >> 4) & 1) == ((lam >> 0) & 1)
    m40_hi = ((lam >> 4) & 1) == 1
    m31_eq = ((lam >> 3) & 1) == ((lam >> 1) & 1)
    m31_hi = ((lam >> 3) & 1) == 1
    b5_hi = ((lam >> 5) & 1) == 1
    b6_hi = ((lam >> 6) & 1) == 1

    def swap_lanes(v, d, m_eq, m_hi):
        vp = pltpu.roll(v, d, axis=1)
        vm = pltpu.roll(v, 128 - d, axis=1)
        return jnp.where(m_eq, v, jnp.where(m_hi, vp, vm))

    def swap_col(lo, hi, d, m_hi):
        new_lo = jnp.where(m_hi, pltpu.roll(hi, d, axis=1), lo)
        new_hi = jnp.where(m_hi, hi, pltpu.roll(lo, 128 - d, axis=1))
        return new_lo, new_hi

    t = [x_ref[:, 128 * k:128 * (k + 1)] for k in range(8)]
    t = [swap_lanes(v, 15, m40_eq, m40_hi) for v in t]
    t = [swap_lanes(v, 6, m31_eq, m31_hi) for v in t]
    for k in (0, 1, 2, 3):
        t[k], t[k + 4] = swap_col(t[k], t[k + 4], 32, b5_hi)
    for k in (0, 1, 4, 5):
        t[k], t[k + 2] = swap_col(t[k], t[k + 2], 64, b6_hi)
    for k in range(8):
        o_ref[:, 128 * k:128 * (k + 1)] = t[k]


def kernel(tensor):
    n, r, c = tensor.shape
    xf = tensor.reshape(n, r * c)
    bm = 1024
    out = pl.pallas_call(
        _permute_body,
        grid=(n // bm,),
        in_specs=[pl.BlockSpec((bm, r * c), lambda i: (i, 0))],
        out_specs=pl.BlockSpec((bm, r * c), lambda i: (i, 0)),
        out_shape=jax.ShapeDtypeStruct((n, r * c), tensor.dtype),
        compiler_params=pltpu.CompilerParams(
            dimension_semantics=("parallel",)),
    )(xf)
    return out.reshape(n, r, c)
